# baseline (speedup 1.0000x reference)
import jax
import jax.numpy as jnp
from jax.experimental import pallas as pl
from jax.experimental.pallas import tpu as pltpu

EMB_D = 64
NUM_E = 1024
N_TOK = 64 * 24 * 24
TILE = 2048
GRID = N_TOK // TILE
COMMIT = 0.25


def _vq_body(x_ref, emb_ref, q_ref, enc_ref, idx_ref, loss_ref, perp_ref,
             counts_scr, e2_scr, colf_scr, losssum_scr):
    i = pl.program_id(0)

    x = x_ref[...]
    emb = emb_ref[...]

    @pl.when(i == 0)
    def _init():
        counts_scr[...] = jnp.zeros_like(counts_scr)
        e2_scr[...] = jnp.sum(emb * emb, axis=1)[None, :]
        colf_scr[...] = jax.lax.broadcasted_iota(
            jnp.int32, (1, NUM_E), 1).astype(jnp.float32)
        losssum_scr[0] = 0.0

    xm2 = x * (-2.0)
    xe2 = jax.lax.dot_general(xm2, emb, (((1,), (1,)), ((), ())),
                              preferred_element_type=jnp.float32)
    x2 = jnp.sum(x * x, axis=1, keepdims=True)
    s = (x2 + xe2) + e2_scr[...]

    smin = jnp.min(s, axis=1, keepdims=True)
    colf = colf_scr[...]
    idxf = jnp.min(jnp.where(s == smin, colf, 2048.0), axis=1,
                   keepdims=True)

    onehot = jnp.where(colf == idxf, 1.0, 0.0)
    enc_ref[...] = onehot
    idx_ref[...] = idxf.astype(jnp.int32)

    q = jax.lax.dot_general(onehot.astype(jnp.bfloat16),
                            emb.astype(jnp.bfloat16), (((1,), (0,)), ((), ())),
                            preferred_element_type=jnp.float32)
    q_ref[...] = q

    losssum_scr[0] += jnp.sum(smin)
    counts_scr[...] += jnp.sum(onehot, axis=0, keepdims=True)

    @pl.when(i == GRID - 1)
    def _final():
        loss = (1.0 + COMMIT) * losssum_scr[0] / (N_TOK * EMB_D)
        loss_ref[...] = jnp.full((1, 1), loss, jnp.float32)
        avg = counts_scr[...] / N_TOK
        perp = jnp.exp(-jnp.sum(avg * jnp.log(avg + 1e-10)))
        perp_ref[...] = jnp.full((1, 1), perp, jnp.float32)


def _vq_call(x, emb):
    return pl.pallas_call(
        _vq_body,
        grid=(GRID,),
        in_specs=[
            pl.BlockSpec((TILE, EMB_D), lambda i: (i, 0)),
            pl.BlockSpec((NUM_E, EMB_D), lambda i: (0, 0)),
        ],
        out_specs=[
            pl.BlockSpec((TILE, EMB_D), lambda i: (i, 0)),
            pl.BlockSpec((TILE, NUM_E), lambda i: (i, 0)),
            pl.BlockSpec((TILE, 1), lambda i: (i, 0)),
            pl.BlockSpec((1, 1), lambda i: (0, 0)),
            pl.BlockSpec((1, 1), lambda i: (0, 0)),
        ],
        out_shape=[
            jax.ShapeDtypeStruct((N_TOK, EMB_D), jnp.float32),
            jax.ShapeDtypeStruct((N_TOK, NUM_E), jnp.float32),
            jax.ShapeDtypeStruct((N_TOK, 1), jnp.int32),
            jax.ShapeDtypeStruct((1, 1), jnp.float32),
            jax.ShapeDtypeStruct((1, 1), jnp.float32),
        ],
        scratch_shapes=[
            pltpu.VMEM((1, NUM_E), jnp.float32),
            pltpu.VMEM((1, NUM_E), jnp.float32),
            pltpu.VMEM((1, NUM_E), jnp.float32),
            pltpu.SMEM((1,), jnp.float32),
        ],
    )(x, emb)


def kernel(inputTensor, emb_weights):
    x = inputTensor.reshape(-1, EMB_D)
    q, enc, idx, loss, perp = _vq_call(x, emb_weights)
    quantized_st = q.reshape(inputTensor.shape)
    encoding_indices = idx.reshape(inputTensor.shape[:-1])
    return (quantized_st, loss[0, 0], perp[0, 0], enc, encoding_indices)


# speedup vs baseline: 1.0344x; 1.0344x over previous
import jax
import jax.numpy as jnp
from jax.experimental import pallas as pl
from jax.experimental.pallas import tpu as pltpu

EMB_D = 64
NUM_E = 1024
N_TOK = 64 * 24 * 24
TILE = 3072
GRID = N_TOK // TILE
COMMIT = 0.25


def _vq_body(x_ref, emb_ref, q_ref, enc_ref, idx_ref, loss_ref, perp_ref,
             counts_scr, e2_scr, colf_scr, losssum_scr):
    i = pl.program_id(0)

    x = x_ref[...]
    emb = emb_ref[...]

    @pl.when(i == 0)
    def _init():
        counts_scr[...] = jnp.zeros_like(counts_scr)
        e2_scr[...] = jnp.sum(emb * emb, axis=1)[None, :]
        colf_scr[...] = jax.lax.broadcasted_iota(
            jnp.int32, (1, NUM_E), 1).astype(jnp.float32)
        losssum_scr[0] = 0.0

    xm2 = x * (-2.0)
    xe2 = jax.lax.dot_general(xm2, emb, (((1,), (1,)), ((), ())),
                              preferred_element_type=jnp.float32)
    x2 = jnp.sum(x * x, axis=1, keepdims=True)
    s = (x2 + xe2) + e2_scr[...]

    smin = jnp.min(s, axis=1, keepdims=True)
    colf = colf_scr[...]
    idxf = jnp.min(jnp.where(s == smin, colf, 2048.0), axis=1,
                   keepdims=True)

    onehot = jnp.where(colf == idxf, 1.0, 0.0)
    enc_ref[...] = onehot
    idx_ref[...] = idxf.astype(jnp.int32)

    q = jax.lax.dot_general(onehot, emb, (((1,), (0,)), ((), ())),
                            preferred_element_type=jnp.float32)
    q_ref[...] = q

    losssum_scr[0] += jnp.sum(smin)
    counts_scr[...] += jnp.sum(onehot, axis=0, keepdims=True)

    @pl.when(i == GRID - 1)
    def _final():
        loss = (1.0 + COMMIT) * losssum_scr[0] / (N_TOK * EMB_D)
        loss_ref[...] = jnp.full((1, 1), loss, jnp.float32)
        avg = counts_scr[...] / N_TOK
        perp = jnp.exp(-jnp.sum(avg * jnp.log(avg + 1e-10)))
        perp_ref[...] = jnp.full((1, 1), perp, jnp.float32)


def _vq_call(x, emb):
    return pl.pallas_call(
        _vq_body,
        grid=(GRID,),
        in_specs=[
            pl.BlockSpec((TILE, EMB_D), lambda i: (i, 0)),
            pl.BlockSpec((NUM_E, EMB_D), lambda i: (0, 0)),
        ],
        out_specs=[
            pl.BlockSpec((TILE, EMB_D), lambda i: (i, 0)),
            pl.BlockSpec((TILE, NUM_E), lambda i: (i, 0)),
            pl.BlockSpec((TILE, 1), lambda i: (i, 0)),
            pl.BlockSpec((1, 1), lambda i: (0, 0)),
            pl.BlockSpec((1, 1), lambda i: (0, 0)),
        ],
        out_shape=[
            jax.ShapeDtypeStruct((N_TOK, EMB_D), jnp.float32),
            jax.ShapeDtypeStruct((N_TOK, NUM_E), jnp.float32),
            jax.ShapeDtypeStruct((N_TOK, 1), jnp.int32),
            jax.ShapeDtypeStruct((1, 1), jnp.float32),
            jax.ShapeDtypeStruct((1, 1), jnp.float32),
        ],
        scratch_shapes=[
            pltpu.VMEM((1, NUM_E), jnp.float32),
            pltpu.VMEM((1, NUM_E), jnp.float32),
            pltpu.VMEM((1, NUM_E), jnp.float32),
            pltpu.SMEM((1,), jnp.float32),
        ],
    )(x, emb)


def kernel(inputTensor, emb_weights):
    x = inputTensor.reshape(-1, EMB_D)
    q, enc, idx, loss, perp = _vq_call(x, emb_weights)
    quantized_st = q.reshape(inputTensor.shape)
    encoding_indices = idx.reshape(inputTensor.shape[:-1])
    return (quantized_st, loss[0, 0], perp[0, 0], enc, encoding_indices)
